# baseline (speedup 1.0000x reference)
import functools

import jax
import jax.numpy as jnp
from jax import lax
from jax.experimental import pallas as pl
from jax.experimental.pallas import tpu as pltpu
from jax.experimental.pallas import tpu_sc as plsc

B = 16384
D = 64
NC = 2
NS = 16
L = 16
NW = NC * NS
BPW = B // NW
CH = 128
NCH = BPW // CH

_mesh = plsc.VectorSubcoreMesh(core_axis_name="c", subcore_axis_name="s")

_params = pltpu.CompilerParams(
    use_tc_tiling_on_sc=False,
    needs_layout_passes=False,
)


@functools.partial(
    pl.kernel,
    mesh=_mesh,
    out_type=jax.ShapeDtypeStruct((NW, BPW), jnp.float32),
    scratch_types=[
        pltpu.VMEM((NCH, CH), jnp.int32),
        pltpu.VMEM((NCH, CH), jnp.int32),
        pltpu.VMEM((BPW, D), jnp.float32),
        pltpu.VMEM((BPW, D), jnp.float32),
        pltpu.VMEM((BPW,), jnp.float32),
        pltpu.SemaphoreType.DMA,
    ],
    compiler_params=_params,
)
def _sc_dot(uid_hbm, bid_hbm, ut_hbm, bt_hbm, out_hbm,
            uidx, bidx, urows, brows, outv, sem):
    wid = lax.axis_index("s") * NC + lax.axis_index("c")

    pltpu.sync_copy(uid_hbm.at[wid], uidx)
    pltpu.sync_copy(bid_hbm.at[wid], bidx)

    copies = []
    for j in range(NCH):
        copies.append(pltpu.async_copy(
            ut_hbm.at[uidx.at[j]], urows.at[pl.ds(j * CH, CH)], sem))
        copies.append(pltpu.async_copy(
            bt_hbm.at[bidx.at[j]], brows.at[pl.ds(j * CH, CH)], sem))
    for c in copies:
        c.wait()

    lane = lax.broadcasted_iota(jnp.int32, (L,), 0)

    def group(g, carry):
        r0 = g * L
        acc = jnp.zeros((L,), jnp.float32)
        for j in range(L):
            r = r0 + j
            s = jnp.zeros((L,), jnp.float32)
            for k in range(D // L):
                u = urows[r, pl.ds(k * L, L)]
                v = brows[r, pl.ds(k * L, L)]
                s = s + u * v
            acc = jnp.where(lane == j, jnp.sum(s), acc)
        outv[pl.ds(r0, L)] = acc
        return carry

    lax.fori_loop(0, BPW // L, group, 0, unroll=False)

    pltpu.sync_copy(outv, out_hbm.at[wid])


@jax.jit
def kernel(user_ids, book_ids, user_table, book_table):
    uid = user_ids.reshape(NW, NCH, CH)
    bid = book_ids.reshape(NW, NCH, CH)
    out = _sc_dot(uid, bid, user_table, book_table)
    return out.reshape(B, 1)
